# baseline (speedup 1.0000x reference)
import functools

import jax
import jax.numpy as jnp
from jax import lax
from jax.experimental import pallas as pl
from jax.experimental.pallas import tpu as pltpu
from jax.experimental.pallas import tpu_sc as plsc

NUM_CLASSES = 1000000
EMBED_DIM = 64
BATCH = 16384

NC = 2
NS = 16
NW = NC * NS
B_PER_W = BATCH // NW
CHUNK = 128
NCH = B_PER_W // CHUNK


def _sc_gather_body(table_hbm, idx_hbm, out_hbm, idx_v, rows_v, sem):
    wid = lax.axis_index("s") * NC + lax.axis_index("c")
    base = wid * NCH
    pltpu.sync_copy(idx_hbm.at[pl.ds(base, NCH)], idx_v)
    copies = [
        pltpu.async_copy(table_hbm.at[idx_v.at[j]], rows_v.at[j], sem)
        for j in range(NCH)
    ]
    for c in copies:
        c.wait()
    pltpu.sync_copy(rows_v, out_hbm.at[pl.ds(base, NCH)])


@jax.jit
def _sc_gather(table, idx2d):
    mesh = plsc.VectorSubcoreMesh(core_axis_name="c", subcore_axis_name="s")
    fn = pl.kernel(
        _sc_gather_body,
        out_type=jax.ShapeDtypeStruct((NW * NCH, CHUNK, EMBED_DIM), jnp.float32),
        mesh=mesh,
        scratch_types=[
            pltpu.VMEM((NCH, CHUNK), jnp.int32),
            pltpu.VMEM((NCH, CHUNK, EMBED_DIM), jnp.float32),
            pltpu.SemaphoreType.DMA,
        ],
        compiler_params=pltpu.CompilerParams(use_tc_tiling_on_sc=False),
    )
    return fn(table, idx2d)


def _tc_linear_body(x_ref, wt_ref, b_ref, o_ref):
    x = x_ref[...]
    s = x * jax.nn.sigmoid(x)
    o_ref[...] = (
        jnp.dot(s, wt_ref[...], preferred_element_type=jnp.float32) + b_ref[...]
    )


@jax.jit
def _tc_linear(x, wt, b2d):
    blk = 2048
    grid = (BATCH // blk,)
    return pl.pallas_call(
        _tc_linear_body,
        grid=grid,
        in_specs=[
            pl.BlockSpec((blk, EMBED_DIM), lambda i: (i, 0)),
            pl.BlockSpec((EMBED_DIM, EMBED_DIM), lambda i: (0, 0)),
            pl.BlockSpec((1, EMBED_DIM), lambda i: (0, 0)),
        ],
        out_specs=pl.BlockSpec((blk, EMBED_DIM), lambda i: (i, 0)),
        out_shape=jax.ShapeDtypeStruct((BATCH, EMBED_DIM), jnp.float32),
    )(x, wt, b2d)


def kernel(class_labels, table, W, b):
    idx2d = class_labels.astype(jnp.int32).reshape(NW * NCH, CHUNK)
    gathered = _sc_gather(table, idx2d)
    x = gathered.reshape(BATCH, EMBED_DIM)
    return _tc_linear(x, W.T, b.reshape(1, EMBED_DIM))


# speedup vs baseline: 2.3745x; 2.3745x over previous
import functools

import jax
import jax.numpy as jnp
from jax import lax
from jax.experimental import pallas as pl
from jax.experimental.pallas import tpu as pltpu
from jax.experimental.pallas import tpu_sc as plsc

NUM_CLASSES = 1000000
EMBED_DIM = 64
BATCH = 16384

NC = 2
NS = 16
NW = NC * NS
B_PER_W = BATCH // NW
CHUNK = 64
NCH = B_PER_W // CHUNK
GRP = 8


def _sc_gather_body(table3, idx_hbm, out_hbm, idx_v, rows_v, sem_g, sem_o):
    wid = lax.axis_index("s") * NC + lax.axis_index("c")
    pltpu.sync_copy(idx_hbm.at[wid], idx_v)

    oh = [None] * NCH
    for j in range(NCH):
        b = j % 2
        gh = []
        if j >= 2:
            oh[j - 2].wait()
        for k in range(CHUNK // 16):
            vec = idx_v[pl.ds(j * CHUNK + k * 16, 16)]
            tv = jnp.right_shift(vec, 3)
            sv = jnp.bitwise_and(vec, GRP - 1)
            for l in range(16):
                gh.append(
                    pltpu.async_copy(
                        table3.at[tv[l], sv[l]],
                        rows_v.at[b, k * 16 + l], sem_g))
        for h in gh:
            h.wait()
        oh[j] = pltpu.async_copy(rows_v.at[b], out_hbm.at[wid, j], sem_o)
    oh[NCH - 2].wait()
    oh[NCH - 1].wait()


@jax.jit
def _sc_gather(table3, idx2d):
    mesh = plsc.VectorSubcoreMesh(core_axis_name="c", subcore_axis_name="s")
    fn = pl.kernel(
        _sc_gather_body,
        out_type=jax.ShapeDtypeStruct((NW, NCH, CHUNK, EMBED_DIM),
                                      jnp.float32),
        mesh=mesh,
        scratch_types=[
            pltpu.VMEM((B_PER_W,), jnp.int32),
            pltpu.VMEM((2, CHUNK, EMBED_DIM), jnp.float32),
            pltpu.SemaphoreType.DMA,
            pltpu.SemaphoreType.DMA,
        ],
    )
    return fn(table3, idx2d)


def _tc_linear_body(x_ref, wt_ref, b_ref, o_ref):
    x = x_ref[...]
    s = x * jax.nn.sigmoid(x)
    o_ref[...] = (
        jnp.dot(s, wt_ref[...], preferred_element_type=jnp.float32) + b_ref[...]
    )


@jax.jit
def _tc_linear(x, wt, b2d):
    blk = 2048
    grid = (BATCH // blk,)
    return pl.pallas_call(
        _tc_linear_body,
        grid=grid,
        in_specs=[
            pl.BlockSpec((blk, EMBED_DIM), lambda i: (i, 0)),
            pl.BlockSpec((EMBED_DIM, EMBED_DIM), lambda i: (0, 0)),
            pl.BlockSpec((1, EMBED_DIM), lambda i: (0, 0)),
        ],
        out_specs=pl.BlockSpec((blk, EMBED_DIM), lambda i: (i, 0)),
        out_shape=jax.ShapeDtypeStruct((BATCH, EMBED_DIM), jnp.float32),
    )(x, wt, b2d)


def kernel(class_labels, table, W, b):
    table3 = table.reshape(NUM_CLASSES // GRP, GRP, EMBED_DIM)
    idx2d = class_labels.astype(jnp.int32).reshape(NW, B_PER_W)
    gathered = _sc_gather(table3, idx2d)
    x = gathered.reshape(BATCH, EMBED_DIM)
    return _tc_linear(x, W.T, b.reshape(1, EMBED_DIM))
